# baseline (device time: 134407 ns/iter reference)
import os

import jax
import jax.numpy as jnp
from jax import lax
from jax.experimental import pallas as pl
from jax.experimental.pallas import tpu as pltpu

N_DEV = 4
NT = 512

_SKIP_COMM = os.environ.get("SKIP_COMM") == "1"


def kernel(x, w_mat):
    x = x.astype(jnp.bfloat16)
    m_per, k = x.shape
    _, n = w_mat.shape
    n_per = n // N_DEV
    tpb = n_per // NT
    n_tiles = N_DEV * tpb
    send_tiles = (N_DEV - 1) * tpb

    def body(x_ref, w_ref, out_ref, send_buf, w_buf, w_sems, send_sems, recv_sems):
        my = lax.axis_index("i")

        barrier = pltpu.get_barrier_semaphore()
        for off in (1, 2, 3):
            peer = lax.rem(my + off, N_DEV)
            pl.semaphore_signal(
                barrier, inc=1,
                device_id=(peer,), device_id_type=pl.DeviceIdType.MESH,
            )
        pl.semaphore_wait(barrier, N_DEV - 1)

        def tile_params(idx):
            send_phase = idx < send_tiles
            bi = jnp.where(send_phase, lax.rem(idx, N_DEV - 1), N_DEV - 1)
            t = jnp.where(send_phase, idx // (N_DEV - 1), idx - send_tiles)
            dest = lax.rem(my + bi + 1, N_DEV)
            return bi, t, dest, send_phase

        def w_copy(idx, slot):
            bi, t, dest, send_phase = tile_params(idx)
            col = dest * n_per + t * NT
            return pltpu.make_async_copy(
                w_ref.at[:, pl.ds(col, NT)], w_buf.at[slot], w_sems.at[slot]
            )

        def send_rdma(sb, t):
            _, _, dest, _ = tile_params(sb + t * (N_DEV - 1))
            return pltpu.make_async_remote_copy(
                src_ref=send_buf.at[sb, :, pl.ds(lax.rem(t, 2) * NT, NT)],
                dst_ref=out_ref.at[pl.ds(my * m_per, m_per), pl.ds(t * NT, NT)],
                send_sem=send_sems.at[sb, t],
                recv_sem=recv_sems.at[my, t],
                device_id=(dest,),
                device_id_type=pl.DeviceIdType.MESH,
            )

        w_copy(jnp.int32(0), 0).start()
        w_copy(jnp.int32(1), 1).start()

        def tile_step(idx, carry):
            bi, t, dest, send_phase = tile_params(idx)
            slot = lax.rem(idx, 2)
            sb = jnp.minimum(bi, N_DEV - 2)

            if not _SKIP_COMM:
                @pl.when(jnp.logical_and(send_phase, t >= 2))
                def _():
                    send_rdma(sb, jnp.maximum(t - 2, 0)).wait_send()

            w_copy(idx, slot).wait()
            acc = jnp.dot(
                x_ref[...], w_buf[slot].astype(jnp.bfloat16),
                preferred_element_type=jnp.float32,
            ).astype(jnp.bfloat16)

            @pl.when(jnp.logical_not(send_phase))
            def _():
                out_ref[pl.ds(my * m_per, m_per), pl.ds(t * NT, NT)] = acc

            @pl.when(send_phase)
            def _():
                send_buf[sb, :, pl.ds(lax.rem(t, 2) * NT, NT)] = acc

            @pl.when(send_phase & (not _SKIP_COMM))
            def _():
                send_rdma(sb, t).start()

            @pl.when(idx + 2 < n_tiles)
            def _():
                w_copy(idx + 2, slot).start()

            return carry

        lax.fori_loop(0, n_tiles, tile_step, 0)

        if not _SKIP_COMM:
            for sb in range(N_DEV - 1):
                for t in range(tpb - 2, tpb):
                    send_rdma(jnp.int32(sb), jnp.int32(t)).wait_send()

            for t in range(tpb):
                for off in (3, 2, 1):
                    src = lax.rem(my + off, N_DEV)
                    pltpu.make_async_remote_copy(
                        src_ref=send_buf.at[0, :, pl.ds(0, NT)],
                        dst_ref=out_ref.at[pl.ds(src * m_per, m_per), pl.ds(t * NT, NT)],
                        send_sem=send_sems.at[0, t],
                        recv_sem=recv_sems.at[src, t],
                        device_id=(src,),
                        device_id_type=pl.DeviceIdType.MESH,
                    ).wait_recv()

    return pl.pallas_call(
        body,
        out_shape=jax.ShapeDtypeStruct((N_DEV * m_per, n_per), jnp.bfloat16),
        in_specs=[
            pl.BlockSpec(memory_space=pltpu.MemorySpace.VMEM),
            pl.BlockSpec(memory_space=pl.ANY),
        ],
        out_specs=pl.BlockSpec(memory_space=pltpu.MemorySpace.VMEM),
        scratch_shapes=[
            pltpu.VMEM((N_DEV - 1, m_per, 2 * NT), jnp.bfloat16),
            pltpu.VMEM((2, k, NT), jnp.float32),
            pltpu.SemaphoreType.DMA((2,)),
            pltpu.SemaphoreType.DMA((N_DEV - 1, tpb)),
            pltpu.SemaphoreType.DMA((N_DEV, tpb)),
        ],
        compiler_params=pltpu.CompilerParams(
            collective_id=0,
            vmem_limit_bytes=40 * 1024 * 1024,
        ),
    )(x, w_mat)


# device time: 129388 ns/iter; 1.0388x vs baseline; 1.0388x over previous
import os

import jax
import jax.numpy as jnp
from jax import lax
from jax.experimental import pallas as pl
from jax.experimental.pallas import tpu as pltpu

N_DEV = 4
NT = 256

_SKIP_COMM = os.environ.get("SKIP_COMM") == "1"


def kernel(x, w_mat):
    x = x.astype(jnp.bfloat16)
    m_per, k = x.shape
    _, n = w_mat.shape
    n_per = n // N_DEV
    tpb = n_per // NT
    n_tiles = N_DEV * tpb
    send_tiles = (N_DEV - 1) * tpb

    def body(x_ref, w_ref, out_ref, send_buf, w_buf, w_bf, w_sems,
             send_sems, recv_sems):
        my = lax.axis_index("i")

        barrier = pltpu.get_barrier_semaphore()
        for off in (1, 2, 3):
            peer = lax.rem(my + off, N_DEV)
            pl.semaphore_signal(
                barrier, inc=1,
                device_id=(peer,), device_id_type=pl.DeviceIdType.MESH,
            )
        pl.semaphore_wait(barrier, N_DEV - 1)

        def tile_params(idx):
            send_phase = idx < send_tiles
            bi = jnp.where(send_phase, lax.rem(idx, N_DEV - 1), N_DEV - 1)
            t = jnp.where(send_phase, idx // (N_DEV - 1), idx - send_tiles)
            dest = lax.rem(my + bi + 1, N_DEV)
            return bi, t, dest, send_phase

        def w_copy(idx):
            bi, t, dest, send_phase = tile_params(idx)
            col = dest * n_per + t * NT
            return pltpu.make_async_copy(
                w_ref.at[:, pl.ds(col, NT)], w_buf, w_sems
            )

        w_copy(jnp.int32(0)).start()
        w_copy(jnp.int32(0)).wait()
        w_bf[0] = w_buf[...].astype(jnp.bfloat16)
        w_copy(jnp.int32(1)).start()

        def tile_step(idx, carry):
            bi, t, dest, send_phase = tile_params(idx)
            slot = lax.rem(idx, 2)
            sb = jnp.minimum(bi, N_DEV - 2)

            acc = jnp.dot(
                x_ref[...], w_bf[slot],
                preferred_element_type=jnp.float32,
            ).astype(jnp.bfloat16)

            @pl.when(idx + 1 < n_tiles)
            def _():
                w_copy(idx + 1).wait()

            @pl.when(idx + 1 < n_tiles)
            def _():
                w_bf[1 - slot] = w_buf[...].astype(jnp.bfloat16)

            @pl.when(idx + 2 < n_tiles)
            def _():
                w_copy(idx + 2).start()

            @pl.when(jnp.logical_not(send_phase))
            def _():
                out_ref[pl.ds(my * m_per, m_per), pl.ds(t * NT, NT)] = acc

            @pl.when(send_phase)
            def _():
                send_buf[sb, :, pl.ds(t * NT, NT)] = acc

            @pl.when(send_phase & (not _SKIP_COMM))
            def _():
                pltpu.make_async_remote_copy(
                    src_ref=send_buf.at[sb, :, pl.ds(t * NT, NT)],
                    dst_ref=out_ref.at[pl.ds(my * m_per, m_per), pl.ds(t * NT, NT)],
                    send_sem=send_sems.at[sb, t],
                    recv_sem=recv_sems.at[my, t],
                    device_id=(dest,),
                    device_id_type=pl.DeviceIdType.MESH,
                ).start()

            return carry

        lax.fori_loop(0, n_tiles, tile_step, 0)

        if not _SKIP_COMM:
            for sb in range(N_DEV - 1):
                for t in range(tpb):
                    pltpu.make_async_remote_copy(
                        src_ref=send_buf.at[sb, :, pl.ds(t * NT, NT)],
                        dst_ref=out_ref.at[pl.ds(my * m_per, m_per), pl.ds(t * NT, NT)],
                        send_sem=send_sems.at[sb, t],
                        recv_sem=recv_sems.at[my, t],
                        device_id=(lax.rem(my + 1, N_DEV),),
                        device_id_type=pl.DeviceIdType.MESH,
                    ).wait_send()

            for t in range(tpb):
                for off in (3, 2, 1):
                    src = lax.rem(my + off, N_DEV)
                    pltpu.make_async_remote_copy(
                        src_ref=send_buf.at[0, :, pl.ds(t * NT, NT)],
                        dst_ref=out_ref.at[pl.ds(src * m_per, m_per), pl.ds(t * NT, NT)],
                        send_sem=send_sems.at[0, t],
                        recv_sem=recv_sems.at[src, t],
                        device_id=(src,),
                        device_id_type=pl.DeviceIdType.MESH,
                    ).wait_recv()

    return pl.pallas_call(
        body,
        out_shape=jax.ShapeDtypeStruct((N_DEV * m_per, n_per), jnp.bfloat16),
        in_specs=[
            pl.BlockSpec(memory_space=pltpu.MemorySpace.VMEM),
            pl.BlockSpec(memory_space=pl.ANY),
        ],
        out_specs=pl.BlockSpec(memory_space=pltpu.MemorySpace.VMEM),
        scratch_shapes=[
            pltpu.VMEM((N_DEV - 1, m_per, n_per), jnp.bfloat16),
            pltpu.VMEM((k, NT), jnp.float32),
            pltpu.VMEM((2, k, NT), jnp.bfloat16),
            pltpu.SemaphoreType.DMA(()),
            pltpu.SemaphoreType.DMA((N_DEV - 1, tpb)),
            pltpu.SemaphoreType.DMA((N_DEV, tpb)),
        ],
        compiler_params=pltpu.CompilerParams(
            collective_id=0,
            vmem_limit_bytes=40 * 1024 * 1024,
        ),
    )(x, w_mat)


# device time: 128676 ns/iter; 1.0445x vs baseline; 1.0055x over previous
import os

import jax
import jax.numpy as jnp
from jax import lax
from jax.experimental import pallas as pl
from jax.experimental.pallas import tpu as pltpu

N_DEV = 4
NT = 256

_SKIP_COMM = os.environ.get("SKIP_COMM") == "1"


def kernel(x, w_mat):
    x = x.astype(jnp.bfloat16)
    m_per, k = x.shape
    _, n = w_mat.shape
    n_per = n // N_DEV
    tpb = n_per // NT
    n_tiles = N_DEV * tpb
    send_tiles = (N_DEV - 1) * tpb

    def body(x_ref, w_ref, out_ref, send_buf, w_buf, w_sems,
             send_sems, recv_sems):
        my = lax.axis_index("i")

        barrier = pltpu.get_barrier_semaphore()
        for off in (1, 2, 3):
            peer = lax.rem(my + off, N_DEV)
            pl.semaphore_signal(
                barrier, inc=1,
                device_id=(peer,), device_id_type=pl.DeviceIdType.MESH,
            )
        pl.semaphore_wait(barrier, N_DEV - 1)

        def tile_params(idx):
            send_phase = idx < send_tiles
            bi = jnp.where(send_phase, lax.rem(idx, N_DEV - 1), N_DEV - 1)
            t = jnp.where(send_phase, idx // (N_DEV - 1), idx - send_tiles)
            dest = lax.rem(my + bi + 1, N_DEV)
            return bi, t, dest, send_phase

        def w_copy(idx, slot):
            bi, t, dest, send_phase = tile_params(idx)
            col = dest * n_per + t * NT
            return pltpu.make_async_copy(
                w_ref.at[:, pl.ds(col, NT)], w_buf.at[slot], w_sems.at[slot]
            )

        w_copy(jnp.int32(0), 0).start()
        w_copy(jnp.int32(1), 1).start()

        def tile_step(idx, carry):
            bi, t, dest, send_phase = tile_params(idx)
            slot = lax.rem(idx, 2)
            sb = jnp.minimum(bi, N_DEV - 2)

            w_copy(idx, slot).wait()
            acc = jnp.dot(
                x_ref[...], w_buf[slot].astype(jnp.bfloat16),
                preferred_element_type=jnp.float32,
            ).astype(jnp.bfloat16)

            @pl.when(idx + 2 < n_tiles)
            def _():
                w_copy(idx + 2, slot).start()

            @pl.when(jnp.logical_not(send_phase))
            def _():
                out_ref[pl.ds(my * m_per, m_per), pl.ds(t * NT, NT)] = acc

            @pl.when(send_phase)
            def _():
                send_buf[sb, :, pl.ds(t * NT, NT)] = acc

            @pl.when(send_phase & (not _SKIP_COMM))
            def _():
                pltpu.make_async_remote_copy(
                    src_ref=send_buf.at[sb, :, pl.ds(t * NT, NT)],
                    dst_ref=out_ref.at[pl.ds(my * m_per, m_per), pl.ds(t * NT, NT)],
                    send_sem=send_sems.at[sb, t],
                    recv_sem=recv_sems.at[my, t],
                    device_id=(dest,),
                    device_id_type=pl.DeviceIdType.MESH,
                ).start()

            return carry

        lax.fori_loop(0, n_tiles, tile_step, 0)

        if not _SKIP_COMM:
            for sb in range(N_DEV - 1):
                for t in range(tpb):
                    pltpu.make_async_remote_copy(
                        src_ref=send_buf.at[sb, :, pl.ds(t * NT, NT)],
                        dst_ref=out_ref.at[pl.ds(my * m_per, m_per), pl.ds(t * NT, NT)],
                        send_sem=send_sems.at[sb, t],
                        recv_sem=recv_sems.at[my, t],
                        device_id=(lax.rem(my + 1, N_DEV),),
                        device_id_type=pl.DeviceIdType.MESH,
                    ).wait_send()

            for t in range(tpb):
                for off in (3, 2, 1):
                    src = lax.rem(my + off, N_DEV)
                    pltpu.make_async_remote_copy(
                        src_ref=send_buf.at[0, :, pl.ds(t * NT, NT)],
                        dst_ref=out_ref.at[pl.ds(src * m_per, m_per), pl.ds(t * NT, NT)],
                        send_sem=send_sems.at[0, t],
                        recv_sem=recv_sems.at[src, t],
                        device_id=(src,),
                        device_id_type=pl.DeviceIdType.MESH,
                    ).wait_recv()

    return pl.pallas_call(
        body,
        out_shape=jax.ShapeDtypeStruct((N_DEV * m_per, n_per), jnp.bfloat16),
        in_specs=[
            pl.BlockSpec(memory_space=pltpu.MemorySpace.VMEM),
            pl.BlockSpec(memory_space=pl.ANY),
        ],
        out_specs=pl.BlockSpec(memory_space=pltpu.MemorySpace.VMEM),
        scratch_shapes=[
            pltpu.VMEM((N_DEV - 1, m_per, n_per), jnp.bfloat16),
            pltpu.VMEM((2, k, NT), jnp.float32),
            pltpu.SemaphoreType.DMA((2,)),
            pltpu.SemaphoreType.DMA((N_DEV - 1, tpb)),
            pltpu.SemaphoreType.DMA((N_DEV, tpb)),
        ],
        compiler_params=pltpu.CompilerParams(
            collective_id=0,
            vmem_limit_bytes=38 * 1024 * 1024,
        ),
    )(x, w_mat)
